# Initial kernel scaffold; baseline (speedup 1.0000x reference)
import jax
import jax.numpy as jnp
from jax.experimental import pallas as pl


def kernel(x, enc_w1, enc_b1, enc_w2, enc_b2, codebook, dec_w1, dec_b1, dec_w2, dec_b2):
    raise NotImplementedError("write your pallas kernel here")


# speedup vs baseline: 1.0798x; 1.0798x over previous
import functools

import jax
import jax.numpy as jnp
from jax import lax
from jax.experimental import pallas as pl
from jax.experimental.pallas import tpu as pltpu
from jax.experimental.pallas import tpu_sc as plsc

B = 8192
NOTES = 4
PITCH = 88
IN_DIM = NOTES * PITCH
HID = 512
EMBED_DIM = 64
NUM_EMBED = 8192
COMMITMENT_COST = 0.25

TB_E = 512
GE = B // TB_E
TB_A = 256
GA = B // TB_A
TB_C = 512
GC = B // TB_C

_NC = 2
_NS = 16
_NW = _NC * _NS
_BPW = B // _NW


def _bdot(a, b):
    return jnp.dot(a.astype(jnp.bfloat16), b.astype(jnp.bfloat16),
                   preferred_element_type=jnp.float32)


def _encoder_body(x_ref, w1_ref, b1_ref, w2_ref, b2_ref, enc_ref):
    h = jnp.maximum(_bdot(x_ref[...], w1_ref[...]) + b1_ref[...], 0.0)
    enc_ref[...] = _bdot(h, w2_ref[...]) + b2_ref[...]


_encoder = pl.pallas_call(
    _encoder_body,
    grid=(GE,),
    in_specs=[
        pl.BlockSpec((TB_E, IN_DIM), lambda i: (i, 0)),
        pl.BlockSpec((IN_DIM, HID), lambda i: (0, 0)),
        pl.BlockSpec((1, HID), lambda i: (0, 0)),
        pl.BlockSpec((HID, EMBED_DIM), lambda i: (0, 0)),
        pl.BlockSpec((1, EMBED_DIM), lambda i: (0, 0)),
    ],
    out_specs=pl.BlockSpec((TB_E, EMBED_DIM), lambda i: (i, 0)),
    out_shape=jax.ShapeDtypeStruct((B, EMBED_DIM), jnp.float32),
)


_CHUNK = 2048


def _argmin_body(e_ref, sq_ref, cbt_ref, cn_ref, idx_ref):
    s = _bdot(e_ref[...], cbt_ref[...])
    d = sq_ref[...] - 2.0 * s + cn_ref[...]
    accv = None
    accj = None
    for t in range(NUM_EMBED // _CHUNK):
        dt = d[:, t * _CHUNK:(t + 1) * _CHUNK]
        vt = jnp.min(dt, axis=1, keepdims=True)
        iot = lax.broadcasted_iota(jnp.int32, dt.shape, 1) + t * _CHUNK
        jt = jnp.min(jnp.where(dt <= vt, iot, NUM_EMBED), axis=1, keepdims=True)
        vt_b = vt.astype(jnp.bfloat16).astype(jnp.float32)
        if accv is None:
            accv, accj = vt_b, jt
        else:
            repl = vt < accv
            accv = jnp.where(repl, vt_b, accv)
            accj = jnp.where(repl, jt, accj)
    idx_ref[0, 0, :] = accj[:, 0]


_argmin = pl.pallas_call(
    _argmin_body,
    grid=(GA,),
    in_specs=[
        pl.BlockSpec((TB_A, EMBED_DIM), lambda i: (i, 0)),
        pl.BlockSpec((TB_A, 1), lambda i: (i, 0)),
        pl.BlockSpec((EMBED_DIM, NUM_EMBED), lambda i: (0, 0)),
        pl.BlockSpec((1, NUM_EMBED), lambda i: (0, 0)),
    ],
    out_specs=pl.BlockSpec((1, 1, TB_A), lambda i: (i, 0, 0)),
    out_shape=jax.ShapeDtypeStruct((GA, 1, TB_A), jnp.int32),
)


@functools.cache
def _make_sc_gather():
    mesh = plsc.VectorSubcoreMesh(core_axis_name="c", subcore_axis_name="s")

    @functools.partial(
        pl.kernel,
        mesh=mesh,
        compiler_params=pltpu.CompilerParams(use_tc_tiling_on_sc=False),
        out_type=jax.ShapeDtypeStruct((B, EMBED_DIM), jnp.float32),
        scratch_types=[
            pltpu.VMEM((_BPW,), jnp.int32),
            pltpu.VMEM((_BPW, EMBED_DIM), jnp.float32),
            pltpu.SemaphoreType.DMA,
        ],
    )
    def _sc_gather(table_hbm, idx_hbm, out_hbm, idx_v, rows_v, sem):
        wid = lax.axis_index("s") * _NC + lax.axis_index("c")
        base = wid * _BPW
        pltpu.sync_copy(idx_hbm.at[pl.ds(base, _BPW)], idx_v)
        pltpu.async_copy(table_hbm.at[idx_v], rows_v, sem).wait()
        pltpu.sync_copy(rows_v, out_hbm.at[pl.ds(base, _BPW)])

    return _sc_gather


def _dec_body(q_ref, e_ref, w1_ref, b1_ref, w2_ref, b2_ref,
              out_ref, loss_ref):
    @pl.when(pl.program_id(0) == 0)
    def _init():
        loss_ref[...] = jnp.zeros((1, 1), jnp.float32)

    q = q_ref[...]
    g = jnp.maximum(_bdot(q, w1_ref[...]) + b1_ref[...], 0.0)
    out_ref[...] = _bdot(g, w2_ref[...]) + b2_ref[...]
    diff = q - e_ref[...]
    loss_ref[...] += jnp.sum(diff * diff, keepdims=True)


_decode = pl.pallas_call(
    _dec_body,
    grid=(GC,),
    in_specs=[
        pl.BlockSpec((TB_C, EMBED_DIM), lambda i: (i, 0)),
        pl.BlockSpec((TB_C, EMBED_DIM), lambda i: (i, 0)),
        pl.BlockSpec((EMBED_DIM, HID), lambda i: (0, 0)),
        pl.BlockSpec((1, HID), lambda i: (0, 0)),
        pl.BlockSpec((HID, IN_DIM), lambda i: (0, 0)),
        pl.BlockSpec((1, IN_DIM), lambda i: (0, 0)),
    ],
    out_specs=[
        pl.BlockSpec((TB_C, IN_DIM), lambda i: (i, 0)),
        pl.BlockSpec((1, 1), lambda i: (0, 0)),
    ],
    out_shape=[
        jax.ShapeDtypeStruct((B, IN_DIM), jnp.float32),
        jax.ShapeDtypeStruct((1, 1), jnp.float32),
    ],
)


def kernel(x, enc_w1, enc_b1, enc_w2, enc_b2, codebook,
           dec_w1, dec_b1, dec_w2, dec_b2):
    xf = x.reshape(B, IN_DIM)
    enc = _encoder(xf, enc_w1, enc_b1[None, :], enc_w2, enc_b2[None, :])
    sq_e = jnp.sum(enc ** 2, axis=1, keepdims=True)
    cn = jnp.sum(codebook ** 2, axis=1)[None, :]
    idx3 = _argmin(enc, sq_e, codebook.T, cn)
    idx = idx3.reshape(B)
    q = _make_sc_gather()(codebook, idx)
    dec, loss = _decode(q, enc, dec_w1, dec_b1[None, :], dec_w2,
                        dec_b2[None, :])
    vq_loss = (COMMITMENT_COST / (B * EMBED_DIM)) * loss[0, 0]
    return dec.reshape(B, NOTES, PITCH), vq_loss


# speedup vs baseline: 1.1122x; 1.0300x over previous
import functools

import jax
import jax.numpy as jnp
from jax import lax
from jax.experimental import pallas as pl
from jax.experimental.pallas import tpu as pltpu
from jax.experimental.pallas import tpu_sc as plsc

B = 8192
NOTES = 4
PITCH = 88
IN_DIM = NOTES * PITCH
HID = 512
EMBED_DIM = 64
NUM_EMBED = 8192
COMMITMENT_COST = 0.25

TB_E = 512
TB_A = 256
TB_C = 512
_CHUNK = 2048

_NC = 2
_NS = 16
_NW = _NC * _NS


def _bdot(a, b):
    return jnp.dot(a.astype(jnp.bfloat16), b.astype(jnp.bfloat16),
                   preferred_element_type=jnp.float32)


def _encoder_body(x_ref, w1_ref, b1_ref, w2_ref, b2_ref, enc_ref):
    h = jnp.maximum(_bdot(x_ref[...], w1_ref[...]) + b1_ref[...], 0.0)
    enc_ref[...] = _bdot(h, w2_ref[...]) + b2_ref[...]


@functools.cache
def _make_encoder(nb):
    return pl.pallas_call(
        _encoder_body,
        grid=(nb // TB_E,),
        in_specs=[
            pl.BlockSpec((TB_E, IN_DIM), lambda i: (i, 0)),
            pl.BlockSpec((IN_DIM, HID), lambda i: (0, 0)),
            pl.BlockSpec((1, HID), lambda i: (0, 0)),
            pl.BlockSpec((HID, EMBED_DIM), lambda i: (0, 0)),
            pl.BlockSpec((1, EMBED_DIM), lambda i: (0, 0)),
        ],
        out_specs=pl.BlockSpec((TB_E, EMBED_DIM), lambda i: (i, 0)),
        out_shape=jax.ShapeDtypeStruct((nb, EMBED_DIM), jnp.float32),
    )


def _argmin_body(e_ref, sq_ref, cbt_ref, cn_ref, idx_ref):
    s = _bdot(e_ref[...], cbt_ref[...])
    d = sq_ref[...] - 2.0 * s + cn_ref[...]
    accv = None
    accj = None
    for t in range(NUM_EMBED // _CHUNK):
        dt = d[:, t * _CHUNK:(t + 1) * _CHUNK]
        vt = jnp.min(dt, axis=1, keepdims=True)
        iot = lax.broadcasted_iota(jnp.int32, dt.shape, 1) + t * _CHUNK
        jt = jnp.min(jnp.where(dt <= vt, iot, NUM_EMBED), axis=1, keepdims=True)
        vt_b = vt.astype(jnp.bfloat16).astype(jnp.float32)
        if accv is None:
            accv, accj = vt_b, jt
        else:
            repl = vt < accv
            accv = jnp.where(repl, vt_b, accv)
            accj = jnp.where(repl, jt, accj)
    idx_ref[0, 0, :] = accj[:, 0]


@functools.cache
def _make_argmin(nb):
    ga = nb // TB_A
    return pl.pallas_call(
        _argmin_body,
        grid=(ga,),
        in_specs=[
            pl.BlockSpec((TB_A, EMBED_DIM), lambda i: (i, 0)),
            pl.BlockSpec((TB_A, 1), lambda i: (i, 0)),
            pl.BlockSpec((EMBED_DIM, NUM_EMBED), lambda i: (0, 0)),
            pl.BlockSpec((1, NUM_EMBED), lambda i: (0, 0)),
        ],
        out_specs=pl.BlockSpec((1, 1, TB_A), lambda i: (i, 0, 0)),
        out_shape=jax.ShapeDtypeStruct((ga, 1, TB_A), jnp.int32),
    )


@functools.cache
def _make_sc_gather(nb):
    bpw = nb // _NW
    mesh = plsc.VectorSubcoreMesh(core_axis_name="c", subcore_axis_name="s")

    @functools.partial(
        pl.kernel,
        mesh=mesh,
        compiler_params=pltpu.CompilerParams(use_tc_tiling_on_sc=False),
        out_type=jax.ShapeDtypeStruct((nb, EMBED_DIM), jnp.float32),
        scratch_types=[
            pltpu.VMEM((bpw,), jnp.int32),
            pltpu.VMEM((bpw, EMBED_DIM), jnp.float32),
            pltpu.SemaphoreType.DMA,
        ],
    )
    def _sc_gather(table_hbm, idx_hbm, out_hbm, idx_v, rows_v, sem):
        wid = lax.axis_index("s") * _NC + lax.axis_index("c")
        base = wid * bpw
        pltpu.sync_copy(idx_hbm.at[pl.ds(base, bpw)], idx_v)
        pltpu.async_copy(table_hbm.at[idx_v], rows_v, sem).wait()
        pltpu.sync_copy(rows_v, out_hbm.at[pl.ds(base, bpw)])

    return _sc_gather


def _dec_body(q_ref, e_ref, w1_ref, b1_ref, w2_ref, b2_ref,
              out_ref, loss_ref):
    @pl.when(pl.program_id(0) == 0)
    def _init():
        loss_ref[...] = jnp.zeros((1, 1), jnp.float32)

    q = q_ref[...]
    g = jnp.maximum(_bdot(q, w1_ref[...]) + b1_ref[...], 0.0)
    out_ref[...] = _bdot(g, w2_ref[...]) + b2_ref[...]
    diff = q - e_ref[...]
    loss_ref[...] += jnp.sum(diff * diff, keepdims=True)


@functools.cache
def _make_decode(nb):
    return pl.pallas_call(
        _dec_body,
        grid=(nb // TB_C,),
        in_specs=[
            pl.BlockSpec((TB_C, EMBED_DIM), lambda i: (i, 0)),
            pl.BlockSpec((TB_C, EMBED_DIM), lambda i: (i, 0)),
            pl.BlockSpec((EMBED_DIM, HID), lambda i: (0, 0)),
            pl.BlockSpec((1, HID), lambda i: (0, 0)),
            pl.BlockSpec((HID, IN_DIM), lambda i: (0, 0)),
            pl.BlockSpec((1, IN_DIM), lambda i: (0, 0)),
        ],
        out_specs=[
            pl.BlockSpec((TB_C, IN_DIM), lambda i: (i, 0)),
            pl.BlockSpec((1, 1), lambda i: (0, 0)),
        ],
        out_shape=[
            jax.ShapeDtypeStruct((nb, IN_DIM), jnp.float32),
            jax.ShapeDtypeStruct((1, 1), jnp.float32),
        ],
    )


def kernel(x, enc_w1, enc_b1, enc_w2, enc_b2, codebook,
           dec_w1, dec_b1, dec_w2, dec_b2):
    xf = x.reshape(B, IN_DIM)
    cbt = codebook.T
    enc = _make_encoder(B)(xf, enc_w1, enc_b1[None, :], enc_w2,
                           enc_b2[None, :])
    sq_e = jnp.sum(enc ** 2, axis=1, keepdims=True)
    cn = jnp.sum(codebook ** 2, axis=1)[None, :]

    bh = B // 2
    decs, losses = [], []
    for h in range(2):
        enc_h = lax.slice_in_dim(enc, h * bh, (h + 1) * bh)
        sq_h = lax.slice_in_dim(sq_e, h * bh, (h + 1) * bh)
        idx3 = _make_argmin(bh)(enc_h, sq_h, cbt, cn)
        q = _make_sc_gather(bh)(codebook, idx3.reshape(bh))
        dec_h, loss_h = _make_decode(bh)(q, enc_h, dec_w1, dec_b1[None, :],
                                         dec_w2, dec_b2[None, :])
        decs.append(dec_h)
        losses.append(loss_h)
    dec = jnp.concatenate(decs, axis=0)
    loss = losses[0][0, 0] + losses[1][0, 0]
    vq_loss = (COMMITMENT_COST / (B * EMBED_DIM)) * loss
    return dec.reshape(B, NOTES, PITCH), vq_loss


# speedup vs baseline: 1.1129x; 1.0006x over previous
import functools

import jax
import jax.numpy as jnp
from jax import lax
from jax.experimental import pallas as pl
from jax.experimental.pallas import tpu as pltpu
from jax.experimental.pallas import tpu_sc as plsc

B = 8192
NOTES = 4
PITCH = 88
IN_DIM = NOTES * PITCH
HID = 512
EMBED_DIM = 64
NUM_EMBED = 8192
COMMITMENT_COST = 0.25

TB_E = 512
TB_A = 256
TB_C = 512
_CHUNK = 2048

_NC = 2
_NS = 16
_NW = _NC * _NS


def _bdot(a, b):
    return jnp.dot(a.astype(jnp.bfloat16), b.astype(jnp.bfloat16),
                   preferred_element_type=jnp.float32)


def _encoder_body(x_ref, w1_ref, b1_ref, w2_ref, b2_ref, enc_ref):
    h = jnp.maximum(_bdot(x_ref[...], w1_ref[...]) + b1_ref[...], 0.0)
    enc_ref[...] = _bdot(h, w2_ref[...]) + b2_ref[...]


@functools.cache
def _make_encoder(nb):
    return pl.pallas_call(
        _encoder_body,
        grid=(nb // TB_E,),
        in_specs=[
            pl.BlockSpec((TB_E, IN_DIM), lambda i: (i, 0)),
            pl.BlockSpec((IN_DIM, HID), lambda i: (0, 0)),
            pl.BlockSpec((1, HID), lambda i: (0, 0)),
            pl.BlockSpec((HID, EMBED_DIM), lambda i: (0, 0)),
            pl.BlockSpec((1, EMBED_DIM), lambda i: (0, 0)),
        ],
        out_specs=pl.BlockSpec((TB_E, EMBED_DIM), lambda i: (i, 0)),
        out_shape=jax.ShapeDtypeStruct((nb, EMBED_DIM), jnp.float32),
    )


def _argmin_body(e_ref, sq_ref, cbt_ref, cn_ref, idx_ref):
    s = _bdot(e_ref[...], cbt_ref[...])
    d = sq_ref[...] - 2.0 * s + cn_ref[...]
    accv = None
    accj = None
    for t in range(NUM_EMBED // _CHUNK):
        dt = d[:, t * _CHUNK:(t + 1) * _CHUNK]
        vt = jnp.min(dt, axis=1, keepdims=True)
        iot = lax.broadcasted_iota(jnp.int32, dt.shape, 1) + t * _CHUNK
        jt = jnp.min(jnp.where(dt <= vt, iot, NUM_EMBED), axis=1, keepdims=True)
        vt_b = vt.astype(jnp.bfloat16).astype(jnp.float32)
        if accv is None:
            accv, accj = vt_b, jt
        else:
            repl = vt < accv
            accv = jnp.where(repl, vt_b, accv)
            accj = jnp.where(repl, jt, accj)
    idx_ref[0, 0, :] = accj[:, 0]


@functools.cache
def _make_argmin(nb):
    ga = nb // TB_A
    return pl.pallas_call(
        _argmin_body,
        grid=(ga,),
        in_specs=[
            pl.BlockSpec((TB_A, EMBED_DIM), lambda i: (i, 0)),
            pl.BlockSpec((TB_A, 1), lambda i: (i, 0)),
            pl.BlockSpec((EMBED_DIM, NUM_EMBED), lambda i: (0, 0)),
            pl.BlockSpec((1, NUM_EMBED), lambda i: (0, 0)),
        ],
        out_specs=pl.BlockSpec((1, 1, TB_A), lambda i: (i, 0, 0)),
        out_shape=jax.ShapeDtypeStruct((ga, 1, TB_A), jnp.int32),
    )


@functools.cache
def _make_sc_gather(nb):
    bpw = nb // _NW
    mesh = plsc.VectorSubcoreMesh(core_axis_name="c", subcore_axis_name="s")

    @functools.partial(
        pl.kernel,
        mesh=mesh,
        compiler_params=pltpu.CompilerParams(use_tc_tiling_on_sc=False),
        out_type=jax.ShapeDtypeStruct((nb, EMBED_DIM), jnp.float32),
        scratch_types=[
            pltpu.VMEM((bpw,), jnp.int32),
            pltpu.VMEM((bpw, EMBED_DIM), jnp.float32),
            pltpu.SemaphoreType.DMA,
        ],
    )
    def _sc_gather(table_hbm, idx_hbm, out_hbm, idx_v, rows_v, sem):
        wid = lax.axis_index("s") * _NC + lax.axis_index("c")
        base = wid * bpw
        pltpu.sync_copy(idx_hbm.at[pl.ds(base, bpw)], idx_v)
        pltpu.async_copy(table_hbm.at[idx_v], rows_v, sem).wait()
        pltpu.sync_copy(rows_v, out_hbm.at[pl.ds(base, bpw)])

    return _sc_gather


def _dec_body(q_ref, e_ref, w1_ref, b1_ref, w2_ref, b2_ref,
              out_ref, loss_ref):
    @pl.when(pl.program_id(0) == 0)
    def _init():
        loss_ref[...] = jnp.zeros((1, 1), jnp.float32)

    q = q_ref[...]
    g = jnp.maximum(_bdot(q, w1_ref[...]) + b1_ref[...], 0.0)
    out_ref[...] = _bdot(g, w2_ref[...]) + b2_ref[...]
    diff = q - e_ref[...]
    loss_ref[...] += jnp.sum(diff * diff, keepdims=True)


@functools.cache
def _make_decode(nb):
    return pl.pallas_call(
        _dec_body,
        grid=(nb // TB_C,),
        in_specs=[
            pl.BlockSpec((TB_C, EMBED_DIM), lambda i: (i, 0)),
            pl.BlockSpec((TB_C, EMBED_DIM), lambda i: (i, 0)),
            pl.BlockSpec((EMBED_DIM, HID), lambda i: (0, 0)),
            pl.BlockSpec((1, HID), lambda i: (0, 0)),
            pl.BlockSpec((HID, IN_DIM), lambda i: (0, 0)),
            pl.BlockSpec((1, IN_DIM), lambda i: (0, 0)),
        ],
        out_specs=[
            pl.BlockSpec((TB_C, IN_DIM), lambda i: (i, 0)),
            pl.BlockSpec((1, 1), lambda i: (0, 0)),
        ],
        out_shape=[
            jax.ShapeDtypeStruct((nb, IN_DIM), jnp.float32),
            jax.ShapeDtypeStruct((1, 1), jnp.float32),
        ],
    )


def kernel(x, enc_w1, enc_b1, enc_w2, enc_b2, codebook,
           dec_w1, dec_b1, dec_w2, dec_b2):
    xf = x.reshape(B, IN_DIM)
    cbt = codebook.T
    enc = _make_encoder(B)(xf, enc_w1, enc_b1[None, :], enc_w2,
                           enc_b2[None, :])
    sq_e = jnp.sum(enc ** 2, axis=1, keepdims=True)
    cn = jnp.sum(codebook ** 2, axis=1)[None, :]

    bh = B // 4
    decs, losses = [], []
    for h in range(4):
        enc_h = lax.slice_in_dim(enc, h * bh, (h + 1) * bh)
        sq_h = lax.slice_in_dim(sq_e, h * bh, (h + 1) * bh)
        idx3 = _make_argmin(bh)(enc_h, sq_h, cbt, cn)
        q = _make_sc_gather(bh)(codebook, idx3.reshape(bh))
        dec_h, loss_h = _make_decode(bh)(q, enc_h, dec_w1, dec_b1[None, :],
                                         dec_w2, dec_b2[None, :])
        decs.append(dec_h)
        losses.append(loss_h)
    dec = jnp.concatenate(decs, axis=0)
    loss = sum(l[0, 0] for l in losses)
    vq_loss = (COMMITMENT_COST / (B * EMBED_DIM)) * loss
    return dec.reshape(B, NOTES, PITCH), vq_loss
